# Initial kernel scaffold; baseline (speedup 1.0000x reference)
import jax
import jax.numpy as jnp
from jax.experimental import pallas as pl


def kernel(loc, conf, targets, priors):
    raise NotImplementedError("write your pallas kernel here")


# speedup vs baseline: 5.1912x; 5.1912x over previous
import functools

import jax
import jax.numpy as jnp
from jax import lax
from jax.experimental import pallas as pl

_THRESH = 0.5
_VAR0 = 0.1
_VAR1 = 0.2
_NEGPOS = 3


def _match_loss_kernel(tgt_ref, pt_ref, locT_ref, ct_ref, ll_ref):
    b = pl.program_id(0)
    tt = tgt_ref[0]
    T = tt.shape[0]
    cx = pt_ref[0:1, :]
    cy = pt_ref[1:2, :]
    w = pt_ref[2:3, :]
    h = pt_ref[3:4, :]
    px1 = cx - w * 0.5
    py1 = cy - h * 0.5
    px2 = cx + w * 0.5
    py2 = cy + h * 0.5
    tx1 = tt[:, 0:1]
    ty1 = tt[:, 1:2]
    tx2 = tt[:, 2:3]
    ty2 = tt[:, 3:4]
    iw = jnp.maximum(jnp.minimum(tx2, px2) - jnp.maximum(tx1, px1), 0.0)
    ih = jnp.maximum(jnp.minimum(ty2, py2) - jnp.maximum(ty1, py1), 0.0)
    inter = iw * ih
    area_t = (tx2 - tx1) * (ty2 - ty1)
    area_p = (px2 - px1) * (py2 - py1)
    ov = inter / (area_t + area_p - inter)

    bpi = jnp.argmax(ov, axis=1, keepdims=True).astype(jnp.int32)
    iota_p = lax.broadcasted_iota(jnp.int32, (1, ov.shape[1]), 1)
    iota_t = lax.broadcasted_iota(jnp.int32, (T, 1), 0)
    force = bpi == iota_p
    idx_over = jnp.max(jnp.where(force, iota_t, -1), axis=0, keepdims=True)
    forced = idx_over >= 0
    bto = jnp.max(ov, axis=0, keepdims=True)
    bti = jnp.argmax(ov, axis=0, keepdims=True).astype(jnp.int32)
    bti = jnp.where(forced, idx_over, bti)
    bov = jnp.where(forced, 2.0, bto)

    mx1 = jnp.zeros_like(cx)
    my1 = jnp.zeros_like(cx)
    mx2 = jnp.zeros_like(cx)
    my2 = jnp.zeros_like(cx)
    mlab = jnp.zeros_like(cx)
    for t in range(T):
        m = bti == t
        mx1 = jnp.where(m, tt[t:t + 1, 0:1], mx1)
        my1 = jnp.where(m, tt[t:t + 1, 1:2], my1)
        mx2 = jnp.where(m, tt[t:t + 1, 2:3], mx2)
        my2 = jnp.where(m, tt[t:t + 1, 3:4], my2)
        mlab = jnp.where(m, tt[t:t + 1, 4:5], mlab)

    conf_t = jnp.where(bov < _THRESH, 0, mlab.astype(jnp.int32) + 1)
    ct_ref[0] = conf_t

    g_cx = ((mx1 + mx2) * 0.5 - cx) / (_VAR0 * w)
    g_cy = ((my1 + my2) * 0.5 - cy) / (_VAR0 * h)
    g_w = jnp.log((mx2 - mx1) / w) / _VAR1
    g_h = jnp.log((my2 - my1) / h) / _VAR1
    posf = (conf_t > 0).astype(jnp.float32)
    lt = locT_ref[0]
    ll = jnp.zeros((1, 1), jnp.float32)
    for comp, g in enumerate((g_cx, g_cy, g_w, g_h)):
        d = lt[comp:comp + 1, :] - g
        ad = jnp.abs(d)
        sl1 = jnp.where(ad < 1.0, 0.5 * d * d, ad - 0.5)
        ll = ll + jnp.sum(sl1 * posf, keepdims=True)

    @pl.when(b == 0)
    def _():
        ll_ref[...] = jnp.zeros_like(ll_ref)

    ll_ref[...] += ll


def _conf_loss_kernel(conf_ref, idx_ref, out_ref):
    x = conf_ref[...]
    mx = jnp.max(x, axis=1, keepdims=True)
    e = jnp.exp(x - mx)
    lse = jnp.log(jnp.sum(e, axis=1, keepdims=True)) + mx
    idx = idx_ref[...]
    iota_c = lax.broadcasted_iota(jnp.int32, (1, x.shape[1]), 1)
    onehot = idx == iota_c
    gathered = jnp.sum(jnp.where(onehot, x, 0.0), axis=1, keepdims=True)
    out_ref[...] = lse - gathered


def _mining_kernel(lc_ref, ct_ref, lc_out_ref, np_out_ref, *, p_real):
    b = pl.program_id(0)
    v = lc_ref[0]
    ct = ct_ref[0]
    pos = ct > 0
    num_pos = jnp.sum(pos.astype(jnp.int32))
    mined = jnp.where(pos, 0.0, v)
    bits = lax.bitcast_convert_type(mined, jnp.int32)
    k = jnp.minimum(_NEGPOS * num_pos, p_real - 1)

    def body(i, t):
        cand = t | lax.shift_left(jnp.int32(1), jnp.int32(30) - i)
        cnt = jnp.sum((bits >= cand).astype(jnp.int32))
        return jnp.where(cnt >= k, cand, t)

    t = lax.fori_loop(0, 31, body, jnp.int32(0))
    tf = lax.bitcast_convert_type(t, jnp.float32)
    gt = bits > t
    c_gt = jnp.sum(gt.astype(jnp.int32))
    s_gt = jnp.sum(jnp.where(gt, mined, 0.0))
    topk = s_gt + (k - c_gt).astype(jnp.float32) * tf
    topk = jnp.where(k > 0, topk, 0.0)
    lc_b = jnp.sum(jnp.where(pos, v, 0.0)) + topk

    @pl.when(b == 0)
    def _():
        lc_out_ref[...] = jnp.zeros_like(lc_out_ref)
        np_out_ref[...] = jnp.zeros_like(np_out_ref)

    lc_out_ref[...] += lc_b.reshape(1, 1)
    np_out_ref[...] += num_pos.astype(jnp.float32).reshape(1, 1)


def _pick_rows(n):
    for d in (2944, 2048, 1024, 512, 256, 128, 64, 32, 16, 8):
        if n % d == 0:
            return d
    return n


def kernel(loc, conf, targets, priors):
    B, P, _ = loc.shape
    C = conf.shape[-1]
    T = targets.shape[1]
    Pp = ((P + 127) // 128) * 128
    rows3 = Pp // 128

    pt = jnp.transpose(priors, (1, 0))
    if Pp > P:
        padcol = jnp.tile(
            jnp.array([[1000.0], [1000.0], [1.0], [1.0]], jnp.float32),
            (1, Pp - P))
        pt = jnp.concatenate([pt, padcol], axis=1)
    locT = jnp.transpose(loc, (0, 2, 1))
    if Pp > P:
        locT = jnp.pad(locT, ((0, 0), (0, 0), (0, Pp - P)))

    ct_pad, ll_sum = pl.pallas_call(
        _match_loss_kernel,
        grid=(B,),
        in_specs=[
            pl.BlockSpec((1, T, 5), lambda b: (b, 0, 0)),
            pl.BlockSpec((4, Pp), lambda b: (0, 0)),
            pl.BlockSpec((1, 4, Pp), lambda b: (b, 0, 0)),
        ],
        out_specs=[
            pl.BlockSpec((1, 1, Pp), lambda b: (b, 0, 0)),
            pl.BlockSpec((1, 1), lambda b: (0, 0)),
        ],
        out_shape=[
            jax.ShapeDtypeStruct((B, 1, Pp), jnp.int32),
            jax.ShapeDtypeStruct((1, 1), jnp.float32),
        ],
    )(targets, pt, locT)

    ct_flat = ct_pad[:, 0, :P].reshape(B * P, 1)
    conf2 = conf.reshape(B * P, C)
    R = _pick_rows(B * P)
    lc_flat = pl.pallas_call(
        _conf_loss_kernel,
        grid=(B * P // R,),
        in_specs=[
            pl.BlockSpec((R, C), lambda i: (i, 0)),
            pl.BlockSpec((R, 1), lambda i: (i, 0)),
        ],
        out_specs=pl.BlockSpec((R, 1), lambda i: (i, 0)),
        out_shape=jax.ShapeDtypeStruct((B * P, 1), jnp.float32),
    )(conf2, ct_flat)

    lc3 = jnp.pad(lc_flat.reshape(B, P), ((0, 0), (0, Pp - P)))
    lc3 = lc3.reshape(B, rows3, 128)
    ct3 = jnp.pad(ct_flat.reshape(B, P), ((0, 0), (0, Pp - P)))
    ct3 = ct3.reshape(B, rows3, 128)
    lc_sum, np_sum = pl.pallas_call(
        functools.partial(_mining_kernel, p_real=P),
        grid=(B,),
        in_specs=[
            pl.BlockSpec((1, rows3, 128), lambda b: (b, 0, 0)),
            pl.BlockSpec((1, rows3, 128), lambda b: (b, 0, 0)),
        ],
        out_specs=[
            pl.BlockSpec((1, 1), lambda b: (0, 0)),
            pl.BlockSpec((1, 1), lambda b: (0, 0)),
        ],
        out_shape=[
            jax.ShapeDtypeStruct((1, 1), jnp.float32),
            jax.ShapeDtypeStruct((1, 1), jnp.float32),
        ],
    )(lc3, ct3)

    n = jnp.maximum(np_sum[0, 0], 1.0)
    return ll_sum[0, 0] / n, lc_sum[0, 0] / n


# speedup vs baseline: 5.5064x; 1.0607x over previous
import functools

import jax
import jax.numpy as jnp
from jax import lax
from jax.experimental import pallas as pl

_THRESH = 0.5
_VAR0 = 0.1
_VAR1 = 0.2
_NEGPOS = 3


def _match_loss_kernel(tgt_ref, pt_ref, locT_ref, ct_ref, ll_ref):
    b = pl.program_id(0)
    tt = tgt_ref[0]
    T = tt.shape[0]
    cx = pt_ref[0:1, :]
    cy = pt_ref[1:2, :]
    w = pt_ref[2:3, :]
    h = pt_ref[3:4, :]
    px1 = cx - w * 0.5
    py1 = cy - h * 0.5
    px2 = cx + w * 0.5
    py2 = cy + h * 0.5
    tx1 = tt[:, 0:1]
    ty1 = tt[:, 1:2]
    tx2 = tt[:, 2:3]
    ty2 = tt[:, 3:4]
    iw = jnp.maximum(jnp.minimum(tx2, px2) - jnp.maximum(tx1, px1), 0.0)
    ih = jnp.maximum(jnp.minimum(ty2, py2) - jnp.maximum(ty1, py1), 0.0)
    inter = iw * ih
    area_t = (tx2 - tx1) * (ty2 - ty1)
    area_p = (px2 - px1) * (py2 - py1)
    ov = inter / (area_t + area_p - inter)

    bpi = jnp.argmax(ov, axis=1, keepdims=True).astype(jnp.int32)
    iota_p = lax.broadcasted_iota(jnp.int32, (1, ov.shape[1]), 1)
    iota_t = lax.broadcasted_iota(jnp.int32, (T, 1), 0)
    force = bpi == iota_p
    idx_over = jnp.max(jnp.where(force, iota_t, -1), axis=0, keepdims=True)
    forced = idx_over >= 0
    bto = jnp.max(ov, axis=0, keepdims=True)
    bti = jnp.argmax(ov, axis=0, keepdims=True).astype(jnp.int32)
    bti = jnp.where(forced, idx_over, bti)
    bov = jnp.where(forced, 2.0, bto)

    mx1 = jnp.zeros_like(cx)
    my1 = jnp.zeros_like(cx)
    mx2 = jnp.zeros_like(cx)
    my2 = jnp.zeros_like(cx)
    mlab = jnp.zeros_like(cx)
    for t in range(T):
        m = bti == t
        mx1 = jnp.where(m, tt[t:t + 1, 0:1], mx1)
        my1 = jnp.where(m, tt[t:t + 1, 1:2], my1)
        mx2 = jnp.where(m, tt[t:t + 1, 2:3], mx2)
        my2 = jnp.where(m, tt[t:t + 1, 3:4], my2)
        mlab = jnp.where(m, tt[t:t + 1, 4:5], mlab)

    conf_t = jnp.where(bov < _THRESH, 0, mlab.astype(jnp.int32) + 1)
    ct_ref[0] = conf_t

    g_cx = ((mx1 + mx2) * 0.5 - cx) / (_VAR0 * w)
    g_cy = ((my1 + my2) * 0.5 - cy) / (_VAR0 * h)
    g_w = jnp.log((mx2 - mx1) / w) / _VAR1
    g_h = jnp.log((my2 - my1) / h) / _VAR1
    posf = (conf_t > 0).astype(jnp.float32)
    lt = locT_ref[0]
    ll = jnp.zeros((1, 1), jnp.float32)
    for comp, g in enumerate((g_cx, g_cy, g_w, g_h)):
        d = lt[comp:comp + 1, :] - g
        ad = jnp.abs(d)
        sl1 = jnp.where(ad < 1.0, 0.5 * d * d, ad - 0.5)
        ll = ll + jnp.sum(sl1 * posf, keepdims=True)

    @pl.when(b == 0)
    def _():
        ll_ref[...] = jnp.zeros_like(ll_ref)

    ll_ref[...] += ll


def _conf_loss_kernel(conf_ref, idx_ref, out_ref):
    x = conf_ref[...]
    mx = jnp.max(x, axis=1, keepdims=True)
    e = jnp.exp(x - mx)
    lse = jnp.log(jnp.sum(e, axis=1, keepdims=True)) + mx
    idx = idx_ref[...]
    iota_c = lax.broadcasted_iota(jnp.int32, (1, x.shape[1]), 1)
    onehot = idx == iota_c
    gathered = jnp.sum(jnp.where(onehot, x, 0.0), axis=1, keepdims=True)
    out_ref[...] = lse - gathered


def _mining_kernel(lc_ref, ct_ref, lc_out_ref, np_out_ref, *, p_real):
    v = lc_ref[...]
    ct = ct_ref[...]
    pos = ct > 0
    num_pos = jnp.sum(pos.astype(jnp.int32), axis=1, keepdims=True)
    mined = jnp.where(pos, 0.0, v)
    bits = lax.bitcast_convert_type(mined, jnp.int32)
    k = jnp.minimum(_NEGPOS * num_pos, p_real - 1)

    def body(i, t):
        cand = t | lax.shift_left(jnp.int32(1), jnp.int32(30) - i)
        cnt = jnp.sum((bits >= cand).astype(jnp.int32), axis=1,
                      keepdims=True)
        return jnp.where(cnt >= k, cand, t)

    t0 = jnp.zeros_like(k)
    t = lax.fori_loop(0, 31, body, t0)
    tf = lax.bitcast_convert_type(t, jnp.float32)
    gt = bits > t
    c_gt = jnp.sum(gt.astype(jnp.int32), axis=1, keepdims=True)
    s_gt = jnp.sum(jnp.where(gt, mined, 0.0), axis=1, keepdims=True)
    topk = s_gt + (k - c_gt).astype(jnp.float32) * tf
    topk = jnp.where(k > 0, topk, 0.0)
    lc_b = jnp.sum(jnp.where(pos, v, 0.0), axis=1, keepdims=True) + topk

    lc_out_ref[...] = jnp.sum(lc_b, keepdims=True)
    np_out_ref[...] = jnp.sum(num_pos.astype(jnp.float32), keepdims=True)


def _pick_rows(n):
    for d in (2944, 2048, 1024, 512, 256, 128, 64, 32, 16, 8):
        if n % d == 0:
            return d
    return n


def kernel(loc, conf, targets, priors):
    B, P, _ = loc.shape
    C = conf.shape[-1]
    T = targets.shape[1]
    Pp = ((P + 127) // 128) * 128
    rows3 = Pp // 128

    pt = jnp.transpose(priors, (1, 0))
    if Pp > P:
        padcol = jnp.tile(
            jnp.array([[1000.0], [1000.0], [1.0], [1.0]], jnp.float32),
            (1, Pp - P))
        pt = jnp.concatenate([pt, padcol], axis=1)
    locT = jnp.transpose(loc, (0, 2, 1))
    if Pp > P:
        locT = jnp.pad(locT, ((0, 0), (0, 0), (0, Pp - P)))

    ct_pad, ll_sum = pl.pallas_call(
        _match_loss_kernel,
        grid=(B,),
        in_specs=[
            pl.BlockSpec((1, T, 5), lambda b: (b, 0, 0)),
            pl.BlockSpec((4, Pp), lambda b: (0, 0)),
            pl.BlockSpec((1, 4, Pp), lambda b: (b, 0, 0)),
        ],
        out_specs=[
            pl.BlockSpec((1, 1, Pp), lambda b: (b, 0, 0)),
            pl.BlockSpec((1, 1), lambda b: (0, 0)),
        ],
        out_shape=[
            jax.ShapeDtypeStruct((B, 1, Pp), jnp.int32),
            jax.ShapeDtypeStruct((1, 1), jnp.float32),
        ],
    )(targets, pt, locT)

    ct_flat = ct_pad[:, 0, :P].reshape(B * P, 1)
    conf2 = conf.reshape(B * P, C)
    R = _pick_rows(B * P)
    lc_flat = pl.pallas_call(
        _conf_loss_kernel,
        grid=(B * P // R,),
        in_specs=[
            pl.BlockSpec((R, C), lambda i: (i, 0)),
            pl.BlockSpec((R, 1), lambda i: (i, 0)),
        ],
        out_specs=pl.BlockSpec((R, 1), lambda i: (i, 0)),
        out_shape=jax.ShapeDtypeStruct((B * P, 1), jnp.float32),
    )(conf2, ct_flat)

    lc3 = jnp.pad(lc_flat.reshape(B, P), ((0, 0), (0, Pp - P)))
    ct3 = jnp.pad(ct_flat.reshape(B, P), ((0, 0), (0, Pp - P)))
    lc_sum, np_sum = pl.pallas_call(
        functools.partial(_mining_kernel, p_real=P),
        grid=(1,),
        in_specs=[
            pl.BlockSpec((B, Pp), lambda i: (0, 0)),
            pl.BlockSpec((B, Pp), lambda i: (0, 0)),
        ],
        out_specs=[
            pl.BlockSpec((1, 1), lambda i: (0, 0)),
            pl.BlockSpec((1, 1), lambda i: (0, 0)),
        ],
        out_shape=[
            jax.ShapeDtypeStruct((1, 1), jnp.float32),
            jax.ShapeDtypeStruct((1, 1), jnp.float32),
        ],
    )(lc3, ct3)

    n = jnp.maximum(np_sum[0, 0], 1.0)
    return ll_sum[0, 0] / n, lc_sum[0, 0] / n


# speedup vs baseline: 13.2672x; 2.4094x over previous
import functools

import jax
import jax.numpy as jnp
from jax import lax
from jax.experimental import pallas as pl

_THRESH = 0.5
_VAR0 = 0.1
_VAR1 = 0.2
_NEGPOS = 3


def _match_loss_kernel(tgt_ref, pt_ref, locT_ref, ct_ref, ll_ref):
    b = pl.program_id(0)
    tt = tgt_ref[0]
    T = tt.shape[0]
    cx = pt_ref[0:1, :]
    cy = pt_ref[1:2, :]
    w = pt_ref[2:3, :]
    h = pt_ref[3:4, :]
    px1 = cx - w * 0.5
    py1 = cy - h * 0.5
    px2 = cx + w * 0.5
    py2 = cy + h * 0.5
    tx1 = tt[:, 0:1]
    ty1 = tt[:, 1:2]
    tx2 = tt[:, 2:3]
    ty2 = tt[:, 3:4]
    iw = jnp.maximum(jnp.minimum(tx2, px2) - jnp.maximum(tx1, px1), 0.0)
    ih = jnp.maximum(jnp.minimum(ty2, py2) - jnp.maximum(ty1, py1), 0.0)
    inter = iw * ih
    area_t = (tx2 - tx1) * (ty2 - ty1)
    area_p = (px2 - px1) * (py2 - py1)
    ov = inter / (area_t + area_p - inter)

    bpi = jnp.argmax(ov, axis=1, keepdims=True).astype(jnp.int32)
    iota_p = lax.broadcasted_iota(jnp.int32, (1, ov.shape[1]), 1)
    iota_t = lax.broadcasted_iota(jnp.int32, (T, 1), 0)
    force = bpi == iota_p
    idx_over = jnp.max(jnp.where(force, iota_t, -1), axis=0, keepdims=True)
    forced = idx_over >= 0
    bto = jnp.max(ov, axis=0, keepdims=True)
    bti = jnp.argmax(ov, axis=0, keepdims=True).astype(jnp.int32)
    bti = jnp.where(forced, idx_over, bti)
    bov = jnp.where(forced, 2.0, bto)

    sel = (bti == iota_t).astype(jnp.float32)
    mx1 = jnp.sum(sel * tx1, axis=0, keepdims=True)
    my1 = jnp.sum(sel * ty1, axis=0, keepdims=True)
    mx2 = jnp.sum(sel * tx2, axis=0, keepdims=True)
    my2 = jnp.sum(sel * ty2, axis=0, keepdims=True)
    mlab = jnp.sum(sel * tt[:, 4:5], axis=0, keepdims=True)

    conf_t = jnp.where(bov < _THRESH, 0,
                       mlab.astype(jnp.int32) + 1)
    ct_ref[0] = conf_t

    g_cx = ((mx1 + mx2) * 0.5 - cx) / (_VAR0 * w)
    g_cy = ((my1 + my2) * 0.5 - cy) / (_VAR0 * h)
    g_w = jnp.log((mx2 - mx1) / w) / _VAR1
    g_h = jnp.log((my2 - my1) / h) / _VAR1
    posf = (conf_t > 0).astype(jnp.float32)
    lt = locT_ref[0]
    ll = jnp.zeros((1, 1), jnp.float32)
    for comp, g in enumerate((g_cx, g_cy, g_w, g_h)):
        d = lt[comp:comp + 1, :] - g
        ad = jnp.abs(d)
        sl1 = jnp.where(ad < 1.0, 0.5 * d * d, ad - 0.5)
        ll = ll + jnp.sum(sl1 * posf, keepdims=True)

    @pl.when(b == 0)
    def _():
        ll_ref[...] = jnp.zeros_like(ll_ref)

    ll_ref[...] += ll


def _conf_loss_kernel(conf_ref, idx_ref, out_ref):
    x = conf_ref[...]
    C = x.shape[0]
    mx = jnp.max(x, axis=0, keepdims=True)
    e = jnp.exp(x - mx)
    lse = jnp.log(jnp.sum(e, axis=0, keepdims=True)) + mx
    idx = idx_ref[...]
    iota_c = lax.broadcasted_iota(jnp.int32, (C, 1), 0)
    onehot = idx == iota_c
    gathered = jnp.sum(jnp.where(onehot, x, 0.0), axis=0, keepdims=True)
    out_ref[...] = lse - gathered


def _mining_kernel(lc_ref, ct_ref, lc_out_ref, np_out_ref, *, p_real):
    v = lc_ref[...]
    ct = ct_ref[...]
    pos = ct > 0
    num_pos = jnp.sum(pos.astype(jnp.int32), axis=1, keepdims=True)
    mined = jnp.where(pos, 0.0, v)
    bits = lax.bitcast_convert_type(mined, jnp.int32)
    k = jnp.minimum(_NEGPOS * num_pos, p_real - 1)

    def body(i, t):
        cand = t | lax.shift_left(jnp.int32(1), jnp.int32(30) - i)
        cnt = jnp.sum((bits >= cand).astype(jnp.int32), axis=1,
                      keepdims=True)
        return jnp.where(cnt >= k, cand, t)

    t0 = jnp.zeros_like(k)
    t = lax.fori_loop(0, 31, body, t0)
    tf = lax.bitcast_convert_type(t, jnp.float32)
    gt = bits > t
    c_gt = jnp.sum(gt.astype(jnp.int32), axis=1, keepdims=True)
    s_gt = jnp.sum(jnp.where(gt, mined, 0.0), axis=1, keepdims=True)
    topk = s_gt + (k - c_gt).astype(jnp.float32) * tf
    topk = jnp.where(k > 0, topk, 0.0)
    lc_b = jnp.sum(jnp.where(pos, v, 0.0), axis=1, keepdims=True) + topk

    lc_out_ref[...] = jnp.sum(lc_b, keepdims=True)
    np_out_ref[...] = jnp.sum(num_pos.astype(jnp.float32), keepdims=True)


def _pick_cols(n):
    for d in (8832, 8192, 4096, 2048, 1024, 512, 256, 128):
        if n % d == 0:
            return d
    return n


def kernel(loc, conf, targets, priors):
    B, P, _ = loc.shape
    C = conf.shape[-1]
    T = targets.shape[1]
    Pp = ((P + 127) // 128) * 128

    pt = jnp.transpose(priors, (1, 0))
    if Pp > P:
        padcol = jnp.tile(
            jnp.array([[1000.0], [1000.0], [1.0], [1.0]], jnp.float32),
            (1, Pp - P))
        pt = jnp.concatenate([pt, padcol], axis=1)
    locT = jnp.transpose(loc, (0, 2, 1))
    if Pp > P:
        locT = jnp.pad(locT, ((0, 0), (0, 0), (0, Pp - P)))

    ct_pad, ll_sum = pl.pallas_call(
        _match_loss_kernel,
        grid=(B,),
        in_specs=[
            pl.BlockSpec((1, T, 5), lambda b: (b, 0, 0)),
            pl.BlockSpec((4, Pp), lambda b: (0, 0)),
            pl.BlockSpec((1, 4, Pp), lambda b: (b, 0, 0)),
        ],
        out_specs=[
            pl.BlockSpec((1, 1, Pp), lambda b: (b, 0, 0)),
            pl.BlockSpec((1, 1), lambda b: (0, 0)),
        ],
        out_shape=[
            jax.ShapeDtypeStruct((B, 1, Pp), jnp.int32),
            jax.ShapeDtypeStruct((1, 1), jnp.float32),
        ],
    )(targets, pt, locT)

    ct_flat = ct_pad[:, 0, :P].reshape(1, B * P)
    confT = jnp.transpose(conf.reshape(B * P, C), (1, 0))
    Rl = _pick_cols(B * P)
    lc_flat = pl.pallas_call(
        _conf_loss_kernel,
        grid=(B * P // Rl,),
        in_specs=[
            pl.BlockSpec((C, Rl), lambda i: (0, i)),
            pl.BlockSpec((1, Rl), lambda i: (0, i)),
        ],
        out_specs=pl.BlockSpec((1, Rl), lambda i: (0, i)),
        out_shape=jax.ShapeDtypeStruct((1, B * P), jnp.float32),
    )(confT, ct_flat)

    lc3 = jnp.pad(lc_flat.reshape(B, P), ((0, 0), (0, Pp - P)))
    ct3 = ct_pad.reshape(B, Pp)
    lc_sum, np_sum = pl.pallas_call(
        functools.partial(_mining_kernel, p_real=P),
        grid=(1,),
        in_specs=[
            pl.BlockSpec((B, Pp), lambda i: (0, 0)),
            pl.BlockSpec((B, Pp), lambda i: (0, 0)),
        ],
        out_specs=[
            pl.BlockSpec((1, 1), lambda i: (0, 0)),
            pl.BlockSpec((1, 1), lambda i: (0, 0)),
        ],
        out_shape=[
            jax.ShapeDtypeStruct((1, 1), jnp.float32),
            jax.ShapeDtypeStruct((1, 1), jnp.float32),
        ],
    )(lc3, ct3)

    n = jnp.maximum(np_sum[0, 0], 1.0)
    return ll_sum[0, 0] / n, lc_sum[0, 0] / n
